# Initial kernel scaffold; baseline (speedup 1.0000x reference)
import jax
import jax.numpy as jnp
from jax.experimental import pallas as pl


def kernel(x, edge_index):
    raise NotImplementedError("write your pallas kernel here")


# speedup vs baseline: 8.5054x; 8.5054x over previous
import functools

import jax
import jax.numpy as jnp
from jax import lax
from jax.experimental import pallas as pl
from jax.experimental.pallas import tpu as pltpu
from jax.experimental.pallas import tpu_sc as plsc

NC = 2
NS = 16
LANES = 16
CH = 80
HW = 16


def _mesh():
  return plsc.VectorSubcoreMesh(
      core_axis_name="c", subcore_axis_name="s", num_cores=NC, num_subcores=NS)


def _zero16():
  return jnp.zeros((LANES,), jnp.float32)


def _hist_call(idx_cat, n, nnz):
  per_tile = nnz // NS
  nchunk = per_tile // CH
  nw = 10
  wpt = n // nw

  def body(idx_hbm, out_hbm, idx_v, ones_v, zb_v, acc_sh):
    c = lax.axis_index("c")
    s = lax.axis_index("s")

    for i in range(CH // LANES):
      ones_v[pl.ds(i * LANES, LANES)] = jnp.ones((LANES,), jnp.float32)

    def zfill(i, carry):
      zb_v[pl.ds(i * LANES, LANES)] = jnp.zeros((LANES,), jnp.float32)
      return carry

    lax.fori_loop(0, wpt // LANES, zfill, 0)

    @pl.when(s < nw)
    def _():
      pltpu.sync_copy(zb_v, acc_sh.at[pl.ds(s * wpt, wpt)])

    plsc.subcore_barrier()

    base0 = c * nnz + s * per_tile

    def step(i, carry):
      pltpu.sync_copy(idx_hbm.at[pl.ds(base0 + i * CH, CH)], idx_v)
      pltpu.sync_copy(ones_v, acc_sh.at[idx_v], add=True)
      return carry

    lax.fori_loop(0, nchunk, step, 0)
    plsc.subcore_barrier()

    @pl.when(s < nw)
    def _():
      pltpu.sync_copy(acc_sh.at[pl.ds(s * wpt, wpt)], zb_v)
      pltpu.sync_copy(zb_v, out_hbm.at[pl.ds(c * n + s * wpt, wpt)])

  run = pl.kernel(
      body,
      out_type=jax.ShapeDtypeStruct((2 * n,), jnp.float32),
      mesh=_mesh(),
      scratch_types=[
          pltpu.VMEM((CH,), jnp.int32),
          pltpu.VMEM((CH,), jnp.float32),
          pltpu.VMEM((wpt,), jnp.float32),
          pltpu.VMEM_SHARED((n,), jnp.float32),
      ],
  )
  return run(idx_cat)


def _pass_call(src, gidx, sidx, n, d, nnz):
  ept = nnz // (NC * NS)
  nchunk = ept // CH
  rpt = n // NS
  zr = 128

  def body(src_hbm, gidx_hbm, sidx_hbm, out_hbm,
           gi_v, si_v, rows_v, zb_v, sem, acc_sh):
    c = lax.axis_index("c")
    s = lax.axis_index("s")

    def zfill(i, carry):
      for j in range(d // LANES):
        zb_v[i, pl.ds(j * LANES, LANES)] = _zero16()
      return carry

    lax.fori_loop(0, zr, zfill, 0)
    nfull = rpt // zr
    for k in range(nfull):
      pltpu.sync_copy(zb_v, acc_sh.at[pl.ds(s * rpt + k * zr, zr)])
    rem = rpt - nfull * zr
    if rem:
      pltpu.sync_copy(zb_v.at[pl.ds(0, rem)],
                      acc_sh.at[pl.ds(s * rpt + nfull * zr, rem)])
    plsc.subcore_barrier()

    base0 = (c * NS + s) * ept

    def step(i, carry):
      base = base0 + i * CH
      pltpu.sync_copy(gidx_hbm.at[pl.ds(base, CH)], gi_v)
      pltpu.sync_copy(sidx_hbm.at[pl.ds(base, CH)], si_v)
      pltpu.async_copy(src_hbm.at[gi_v], rows_v, sem).wait()
      pltpu.sync_copy(rows_v, acc_sh.at[si_v], add=True)
      return carry

    lax.fori_loop(0, nchunk, step, 0)
    plsc.subcore_barrier()
    pltpu.sync_copy(acc_sh.at[pl.ds(s * rpt, rpt)], out_hbm.at[c, s])

  run = pl.kernel(
      body,
      out_type=jax.ShapeDtypeStruct((NC, NS, n // NS, d), jnp.float32),
      mesh=_mesh(),
      scratch_types=[
          pltpu.VMEM((CH,), jnp.int32),
          pltpu.VMEM((CH,), jnp.int32),
          pltpu.VMEM((CH, d), jnp.float32),
          pltpu.VMEM((zr, d), jnp.float32),
          pltpu.SemaphoreType.DMA,
          pltpu.VMEM_SHARED((n, d), jnp.float32),
      ],
  )
  return run(src, gidx, sidx)


def _merge_call(partials, cnt, n, d, blk=1000):
  def body(p_ref, c_ref, o_ref):
    cnt0 = c_ref[...]
    scale = jnp.where(cnt0 > 0, 1.0 / cnt0, 0.0)
    o_ref[:, :] = (p_ref[0] + p_ref[1]) * scale

  return pl.pallas_call(
      body,
      grid=(n // blk,),
      in_specs=[
          pl.BlockSpec((2, blk, d), lambda i: (0, i, 0)),
          pl.BlockSpec((blk, 1), lambda i: (i, 0)),
      ],
      out_specs=pl.BlockSpec((blk, d), lambda i: (i, 0)),
      out_shape=jax.ShapeDtypeStruct((n, d), jnp.float32),
  )(partials, cnt)


def _final_call(partials, cnt, x, h1, n, d, blk=1000):
  def body(p_ref, c_ref, x_ref, h_ref, o_ref):
    cnt0 = c_ref[...]
    scale = jnp.where(cnt0 > 0, 1.0 / cnt0, 0.0)
    h2 = (p_ref[0] + p_ref[1]) * scale
    o_ref[:, :] = (x_ref[:, :] + h_ref[:, :] + h2) * jnp.float32(1.0 / 3.0)

  return pl.pallas_call(
      body,
      grid=(n // blk,),
      in_specs=[
          pl.BlockSpec((2, blk, d), lambda i: (0, i, 0)),
          pl.BlockSpec((blk, 1), lambda i: (i, 0)),
          pl.BlockSpec((blk, d), lambda i: (i, 0)),
          pl.BlockSpec((blk, d), lambda i: (i, 0)),
      ],
      out_specs=pl.BlockSpec((blk, d), lambda i: (i, 0)),
      out_shape=jax.ShapeDtypeStruct((n, d), jnp.float32),
  )(partials, cnt, x, h1)


def kernel(x, edge_index):
  n, d = x.shape
  nnz = edge_index.shape[1]
  assert n % NS == 0 and nnz % (NC * NS * CH) == 0 and d % LANES == 0

  row = edge_index[0].astype(jnp.int32)
  col = edge_index[1].astype(jnp.int32)

  hist = _hist_call(jnp.concatenate([row, col]), n, nnz).reshape(2, n, 1)
  cnt_row = hist[0]
  cnt_col = hist[1]

  p1 = _pass_call(x, row, col, n, d, nnz).reshape(2, n, d)
  out_e1 = _merge_call(p1, cnt_col, n, d)
  p2 = _pass_call(out_e1, col, row, n, d, nnz).reshape(2, n, d)
  h1 = _merge_call(p2, cnt_row, n, d)

  p3 = _pass_call(h1, row, col, n, d, nnz).reshape(2, n, d)
  out_e2 = _merge_call(p3, cnt_col, n, d)
  p4 = _pass_call(out_e2, col, row, n, d, nnz).reshape(2, n, d)
  return _final_call(p4, cnt_row, x, h1, n, d)


# speedup vs baseline: 12.9680x; 1.5247x over previous
import functools

import jax
import jax.numpy as jnp
from jax import lax
from jax.experimental import pallas as pl
from jax.experimental.pallas import tpu as pltpu
from jax.experimental.pallas import tpu_sc as plsc

NC = 2
NS = 16
LANES = 16
CH = 80
HW = 16


def _mesh():
  return plsc.VectorSubcoreMesh(
      core_axis_name="c", subcore_axis_name="s", num_cores=NC, num_subcores=NS)


def _zero16():
  return jnp.zeros((LANES,), jnp.float32)


def _hist_call(idx_cat, n, nnz):
  per_tile = nnz // NS
  nchunk = per_tile // CH
  nw = 10
  wpt = n // nw

  def body(idx_hbm, out_hbm, idx_v, ones_v, zb_v, acc_sh):
    c = lax.axis_index("c")
    s = lax.axis_index("s")

    for i in range(CH // LANES):
      ones_v[pl.ds(i * LANES, LANES)] = jnp.ones((LANES,), jnp.float32)

    def zfill(i, carry):
      zb_v[pl.ds(i * LANES, LANES)] = jnp.zeros((LANES,), jnp.float32)
      return carry

    lax.fori_loop(0, wpt // LANES, zfill, 0)

    @pl.when(s < nw)
    def _():
      pltpu.sync_copy(zb_v, acc_sh.at[pl.ds(s * wpt, wpt)])

    plsc.subcore_barrier()

    base0 = c * nnz + s * per_tile

    def step(i, carry):
      pltpu.sync_copy(idx_hbm.at[pl.ds(base0 + i * CH, CH)], idx_v)
      pltpu.sync_copy(ones_v, acc_sh.at[idx_v], add=True)
      return carry

    lax.fori_loop(0, nchunk, step, 0)
    plsc.subcore_barrier()

    @pl.when(s < nw)
    def _():
      pltpu.sync_copy(acc_sh.at[pl.ds(s * wpt, wpt)], zb_v)
      pltpu.sync_copy(zb_v, out_hbm.at[pl.ds(c * n + s * wpt, wpt)])

  run = pl.kernel(
      body,
      out_type=jax.ShapeDtypeStruct((2 * n,), jnp.float32),
      mesh=_mesh(),
      scratch_types=[
          pltpu.VMEM((CH,), jnp.int32),
          pltpu.VMEM((CH,), jnp.float32),
          pltpu.VMEM((wpt,), jnp.float32),
          pltpu.VMEM_SHARED((n,), jnp.float32),
      ],
  )
  return run(idx_cat)


def _pass_call(src, gidx, sidx, n, d, nnz):
  ept = nnz // (NC * NS)
  nchunk = ept // CH
  rpt = n // NS
  zr = 128

  assert nchunk % 2 == 1 and nchunk >= 3

  def body(src_hbm, gidx_hbm, sidx_hbm, out_hbm,
           gi0, si0, rows0, gi1, si1, rows1, zb_v, sem0, sem1, acc_sh):
    c = lax.axis_index("c")
    s = lax.axis_index("s")
    gi = (gi0, gi1)
    si = (si0, si1)
    rows = (rows0, rows1)
    sem = (sem0, sem1)

    def zfill(i, carry):
      for j in range(d // LANES):
        zb_v[i, pl.ds(j * LANES, LANES)] = _zero16()
      return carry

    lax.fori_loop(0, zr, zfill, 0)
    nfull = rpt // zr
    for k in range(nfull):
      pltpu.sync_copy(zb_v, acc_sh.at[pl.ds(s * rpt + k * zr, zr)])
    rem = rpt - nfull * zr
    if rem:
      pltpu.sync_copy(zb_v.at[pl.ds(0, rem)],
                      acc_sh.at[pl.ds(s * rpt + nfull * zr, rem)])
    plsc.subcore_barrier()

    base0 = (c * NS + s) * ept

    def load_idx(chunk, b):
      base = base0 + chunk * CH
      pltpu.sync_copy(gidx_hbm.at[pl.ds(base, CH)], gi[b])
      pltpu.sync_copy(sidx_hbm.at[pl.ds(base, CH)], si[b])

    def start_gather(b):
      pltpu.async_copy(src_hbm.at[gi[b]], rows[b], sem[b])

    def wait_gather(b):
      pltpu.make_async_copy(src_hbm.at[gi[b]], rows[b], sem[b]).wait()

    def scatter(b):
      pltpu.sync_copy(rows[b], acc_sh.at[si[b]], add=True)

    load_idx(0, 0)
    start_gather(0)

    def step(j, carry):
      a = 2 * j
      load_idx(a + 1, 1)
      start_gather(1)
      wait_gather(0)
      scatter(0)
      load_idx(a + 2, 0)
      start_gather(0)
      wait_gather(1)
      scatter(1)
      return carry

    lax.fori_loop(0, (nchunk - 1) // 2, step, 0)
    wait_gather(0)
    scatter(0)

    plsc.subcore_barrier()
    pltpu.sync_copy(acc_sh.at[pl.ds(s * rpt, rpt)], out_hbm.at[c, s])

  run = pl.kernel(
      body,
      out_type=jax.ShapeDtypeStruct((NC, NS, n // NS, d), jnp.float32),
      mesh=_mesh(),
      scratch_types=[
          pltpu.VMEM((CH,), jnp.int32),
          pltpu.VMEM((CH,), jnp.int32),
          pltpu.VMEM((CH, d), jnp.float32),
          pltpu.VMEM((CH,), jnp.int32),
          pltpu.VMEM((CH,), jnp.int32),
          pltpu.VMEM((CH, d), jnp.float32),
          pltpu.VMEM((zr, d), jnp.float32),
          pltpu.SemaphoreType.DMA,
          pltpu.SemaphoreType.DMA,
          pltpu.VMEM_SHARED((n, d), jnp.float32),
      ],
  )
  return run(src, gidx, sidx)


def _merge_call(partials, cnt, n, d, blk=1000):
  def body(p_ref, c_ref, o_ref):
    cnt0 = c_ref[...]
    scale = jnp.where(cnt0 > 0, 1.0 / cnt0, 0.0)
    o_ref[:, :] = (p_ref[0] + p_ref[1]) * scale

  return pl.pallas_call(
      body,
      grid=(n // blk,),
      in_specs=[
          pl.BlockSpec((2, blk, d), lambda i: (0, i, 0)),
          pl.BlockSpec((blk, 1), lambda i: (i, 0)),
      ],
      out_specs=pl.BlockSpec((blk, d), lambda i: (i, 0)),
      out_shape=jax.ShapeDtypeStruct((n, d), jnp.float32),
  )(partials, cnt)


def _final_call(partials, cnt, x, h1, n, d, blk=1000):
  def body(p_ref, c_ref, x_ref, h_ref, o_ref):
    cnt0 = c_ref[...]
    scale = jnp.where(cnt0 > 0, 1.0 / cnt0, 0.0)
    h2 = (p_ref[0] + p_ref[1]) * scale
    o_ref[:, :] = (x_ref[:, :] + h_ref[:, :] + h2) * jnp.float32(1.0 / 3.0)

  return pl.pallas_call(
      body,
      grid=(n // blk,),
      in_specs=[
          pl.BlockSpec((2, blk, d), lambda i: (0, i, 0)),
          pl.BlockSpec((blk, 1), lambda i: (i, 0)),
          pl.BlockSpec((blk, d), lambda i: (i, 0)),
          pl.BlockSpec((blk, d), lambda i: (i, 0)),
      ],
      out_specs=pl.BlockSpec((blk, d), lambda i: (i, 0)),
      out_shape=jax.ShapeDtypeStruct((n, d), jnp.float32),
  )(partials, cnt, x, h1)


def kernel(x, edge_index):
  n, d = x.shape
  nnz = edge_index.shape[1]
  assert n % NS == 0 and nnz % (NC * NS * CH) == 0 and d % LANES == 0

  row = edge_index[0].astype(jnp.int32)
  col = edge_index[1].astype(jnp.int32)

  hist = _hist_call(jnp.concatenate([row, col]), n, nnz).reshape(2, n, 1)
  cnt_row = hist[0]
  cnt_col = hist[1]

  p1 = _pass_call(x, row, col, n, d, nnz).reshape(2, n, d)
  out_e1 = _merge_call(p1, cnt_col, n, d)
  p2 = _pass_call(out_e1, col, row, n, d, nnz).reshape(2, n, d)
  h1 = _merge_call(p2, cnt_row, n, d)

  p3 = _pass_call(h1, row, col, n, d, nnz).reshape(2, n, d)
  out_e2 = _merge_call(p3, cnt_col, n, d)
  p4 = _pass_call(out_e2, col, row, n, d, nnz).reshape(2, n, d)
  return _final_call(p4, cnt_row, x, h1, n, d)


# speedup vs baseline: 16.6146x; 1.2812x over previous
import functools

import jax
import jax.numpy as jnp
from jax import lax
from jax.experimental import pallas as pl
from jax.experimental.pallas import tpu as pltpu
from jax.experimental.pallas import tpu_sc as plsc

NC = 2
NS = 16
LANES = 16
CH = 40
HW = 16


def _mesh():
  return plsc.VectorSubcoreMesh(
      core_axis_name="c", subcore_axis_name="s", num_cores=NC, num_subcores=NS)


def _zero16():
  return jnp.zeros((LANES,), jnp.float32)


def _hist_call(idx_cat, n, nnz):
  chh = 80
  per_tile = nnz // NS
  nchunk = per_tile // chh
  nw = 10
  wpt = n // nw

  def body(idx_hbm, out_hbm, idx0, idx1, isem0, isem1, ones_v, zb_v, acc_sh):
    idx = (idx0, idx1)
    isem = (isem0, isem1)
    c = lax.axis_index("c")
    s = lax.axis_index("s")
    base0 = c * nnz + s * per_tile

    def idx_start(k, r):
      pltpu.async_copy(idx_hbm.at[pl.ds(base0 + k * chh, chh)], idx[r],
                       isem[r])

    def idx_wait(k, r):
      pltpu.make_async_copy(idx_hbm.at[pl.ds(base0 + k * chh, chh)], idx[r],
                            isem[r]).wait()

    def scat(r):
      pltpu.sync_copy(ones_v, acc_sh.at[idx[r]], add=True)

    for i in range(chh // LANES):
      ones_v[pl.ds(i * LANES, LANES)] = jnp.ones((LANES,), jnp.float32)

    def zfill(i, carry):
      zb_v[pl.ds(i * LANES, LANES)] = jnp.zeros((LANES,), jnp.float32)
      return carry

    lax.fori_loop(0, wpt // LANES, zfill, 0)

    @pl.when(s < nw)
    def _():
      pltpu.sync_copy(zb_v, acc_sh.at[pl.ds(s * wpt, wpt)])

    idx_start(0, 0)
    plsc.subcore_barrier()

    def step(j, carry):
      a = 2 * j
      idx_wait(a, 0)
      idx_start(a + 1, 1)
      scat(0)
      idx_wait(a + 1, 1)

      @pl.when(j < nchunk // 2 - 1)
      def _():
        idx_start(a + 2, 0)

      scat(1)
      return carry

    lax.fori_loop(0, nchunk // 2, step, 0)
    plsc.subcore_barrier()

    @pl.when(s < nw)
    def _():
      pltpu.sync_copy(acc_sh.at[pl.ds(s * wpt, wpt)], zb_v)
      pltpu.sync_copy(zb_v, out_hbm.at[pl.ds(c * n + s * wpt, wpt)])

  run = pl.kernel(
      body,
      out_type=jax.ShapeDtypeStruct((2 * n,), jnp.float32),
      mesh=_mesh(),
      scratch_types=[
          pltpu.VMEM((chh,), jnp.int32),
          pltpu.VMEM((chh,), jnp.int32),
          pltpu.SemaphoreType.DMA,
          pltpu.SemaphoreType.DMA,
          pltpu.VMEM((chh,), jnp.float32),
          pltpu.VMEM((wpt,), jnp.float32),
          pltpu.VMEM_SHARED((n,), jnp.float32),
      ],
  )
  return run(idx_cat)


def _pass_call(src, gidx, sidx, n, d, nnz):
  ept = nnz // (NC * NS)
  nchunk = ept // CH
  rpt = n // NS
  zr = 40
  R = 5

  assert nchunk % R == 0 and nchunk // R >= 2

  def body(src_hbm, gidx_hbm, sidx_hbm, out_hbm, *scr):
    gi = scr[0:R]
    si = scr[R:2 * R]
    rows = scr[2 * R:3 * R]
    isem = scr[3 * R:4 * R]
    gsem = scr[4 * R:5 * R]
    ssem = scr[5 * R:6 * R]
    zb_v = scr[6 * R]
    acc_sh = scr[6 * R + 1]
    c = lax.axis_index("c")
    s = lax.axis_index("s")
    base0 = (c * NS + s) * ept

    def idx_start(k, r):
      base = base0 + k * CH
      pltpu.async_copy(gidx_hbm.at[pl.ds(base, CH)], gi[r], isem[r])
      pltpu.async_copy(sidx_hbm.at[pl.ds(base, CH)], si[r], isem[r])

    def idx_wait(k, r):
      base = base0 + k * CH
      pltpu.make_async_copy(gidx_hbm.at[pl.ds(base, CH)], gi[r],
                            isem[r]).wait()
      pltpu.make_async_copy(sidx_hbm.at[pl.ds(base, CH)], si[r],
                            isem[r]).wait()

    def gather_start(r):
      pltpu.async_copy(src_hbm.at[gi[r]], rows[r], gsem[r])

    def gather_wait(r):
      pltpu.make_async_copy(src_hbm.at[gi[r]], rows[r], gsem[r]).wait()

    def scat_start(r):
      pltpu.make_async_copy(rows[r], acc_sh.at[si[r]], ssem[r]).start(add=True)

    def scat_wait(r):
      pltpu.make_async_copy(rows[r], acc_sh.at[si[r]], ssem[r]).wait()

    def zfill(i, carry):
      for j in range(d // LANES):
        zb_v[i, pl.ds(j * LANES, LANES)] = _zero16()
      return carry

    lax.fori_loop(0, zr, zfill, 0)
    nfull = rpt // zr
    for k in range(nfull):
      pltpu.sync_copy(zb_v, acc_sh.at[pl.ds(s * rpt + k * zr, zr)])
    rem = rpt - nfull * zr
    if rem:
      pltpu.sync_copy(zb_v.at[pl.ds(0, rem)],
                      acc_sh.at[pl.ds(s * rpt + nfull * zr, rem)])

    idx_start(0, 0)
    idx_start(1, 1)
    idx_wait(0, 0); gather_start(0)
    idx_start(2, 2)
    idx_wait(1, 1); gather_start(1)
    plsc.subcore_barrier()
    gather_wait(0); scat_start(0)
    idx_start(3, 3)
    idx_wait(2, 2); gather_start(2)
    gather_wait(1); scat_start(1)
    idx_start(4, 4)
    idx_wait(3, 3); gather_start(3)
    gather_wait(2); scat_start(2)

    def lstep(j, carry):
      kb = R * j
      for r in range(R):
        k = kb + r
        scat_wait(r)
        idx_start(k, r)
        r1 = (r - 1) % R
        idx_wait(k - 1, r1)
        gather_start(r1)
        r2 = (r - 2) % R
        gather_wait(r2)
        scat_start(r2)
      return carry

    lax.fori_loop(1, nchunk // R, lstep, 0)

    idx_wait(nchunk - 1, (nchunk - 1) % R)
    gather_start((nchunk - 1) % R)
    gather_wait((nchunk - 2) % R)
    scat_start((nchunk - 2) % R)
    gather_wait((nchunk - 1) % R)
    scat_start((nchunk - 1) % R)
    for r in range(R):
      scat_wait(r)

    plsc.subcore_barrier()
    pltpu.sync_copy(acc_sh.at[pl.ds(s * rpt, rpt)], out_hbm.at[c, s])

  run = pl.kernel(
      body,
      out_type=jax.ShapeDtypeStruct((NC, NS, n // NS, d), jnp.float32),
      mesh=_mesh(),
      scratch_types=(
          [pltpu.VMEM((CH,), jnp.int32)] * R
          + [pltpu.VMEM((CH,), jnp.int32)] * R
          + [pltpu.VMEM((CH, d), jnp.float32)] * R
          + [pltpu.SemaphoreType.DMA] * (3 * R)
          + [pltpu.VMEM((zr, d), jnp.float32),
             pltpu.VMEM_SHARED((n, d), jnp.float32)]
      ),
  )
  return run(src, gidx, sidx)


def _merge_call(partials, cnt, n, d, blk=1000):
  def body(p_ref, c_ref, o_ref):
    cnt0 = c_ref[...]
    scale = jnp.where(cnt0 > 0, 1.0 / cnt0, 0.0)
    o_ref[:, :] = (p_ref[0] + p_ref[1]) * scale

  return pl.pallas_call(
      body,
      grid=(n // blk,),
      in_specs=[
          pl.BlockSpec((2, blk, d), lambda i: (0, i, 0)),
          pl.BlockSpec((blk, 1), lambda i: (i, 0)),
      ],
      out_specs=pl.BlockSpec((blk, d), lambda i: (i, 0)),
      out_shape=jax.ShapeDtypeStruct((n, d), jnp.float32),
  )(partials, cnt)


def _final_call(partials, cnt, x, h1, n, d, blk=1000):
  def body(p_ref, c_ref, x_ref, h_ref, o_ref):
    cnt0 = c_ref[...]
    scale = jnp.where(cnt0 > 0, 1.0 / cnt0, 0.0)
    h2 = (p_ref[0] + p_ref[1]) * scale
    o_ref[:, :] = (x_ref[:, :] + h_ref[:, :] + h2) * jnp.float32(1.0 / 3.0)

  return pl.pallas_call(
      body,
      grid=(n // blk,),
      in_specs=[
          pl.BlockSpec((2, blk, d), lambda i: (0, i, 0)),
          pl.BlockSpec((blk, 1), lambda i: (i, 0)),
          pl.BlockSpec((blk, d), lambda i: (i, 0)),
          pl.BlockSpec((blk, d), lambda i: (i, 0)),
      ],
      out_specs=pl.BlockSpec((blk, d), lambda i: (i, 0)),
      out_shape=jax.ShapeDtypeStruct((n, d), jnp.float32),
  )(partials, cnt, x, h1)


def kernel(x, edge_index):
  n, d = x.shape
  nnz = edge_index.shape[1]
  assert n % NS == 0 and nnz % (NC * NS * CH) == 0 and d % LANES == 0

  row = edge_index[0].astype(jnp.int32)
  col = edge_index[1].astype(jnp.int32)

  hist = _hist_call(jnp.concatenate([row, col]), n, nnz).reshape(2, n, 1)
  cnt_row = hist[0]
  cnt_col = hist[1]

  p1 = _pass_call(x, row, col, n, d, nnz).reshape(2, n, d)
  out_e1 = _merge_call(p1, cnt_col, n, d)
  p2 = _pass_call(out_e1, col, row, n, d, nnz).reshape(2, n, d)
  h1 = _merge_call(p2, cnt_row, n, d)

  p3 = _pass_call(h1, row, col, n, d, nnz).reshape(2, n, d)
  out_e2 = _merge_call(p3, cnt_col, n, d)
  p4 = _pass_call(out_e2, col, row, n, d, nnz).reshape(2, n, d)
  return _final_call(p4, cnt_row, x, h1, n, d)
